# Initial kernel scaffold; baseline (speedup 1.0000x reference)
import jax
import jax.numpy as jnp
from jax.experimental import pallas as pl


def kernel(x, pe):
    raise NotImplementedError("write your pallas kernel here")


# speedup vs baseline: 1.4250x; 1.4250x over previous
import jax
import jax.numpy as jnp
from jax import lax
from jax.experimental import pallas as pl
from jax.experimental.pallas import tpu as pltpu
from jax.experimental.pallas import tpu_sc as plsc

B, S = 4, 8192
D = 1024
NC, NS = 2, 16
NW = NC * NS
TOTAL = B * S
PER_W = TOTAL // NW
CHUNK = 32
NCH = PER_W // CHUNK


def _body(x_hbm, pe_hbm, out_hbm, idx_v, rows_v, gsem):
    wid = lax.axis_index("s") * NC + lax.axis_index("c")
    base = wid * PER_W

    pltpu.sync_copy(x_hbm.at[wid], idx_v)

    zeros = jnp.zeros((16,), jnp.float32)

    def chunk_body(c, carry):
        pltpu.async_copy(pe_hbm.at[idx_v.at[c]], rows_v, gsem).wait()

        vecs = [idx_v[c, pl.ds(g * 16, 16)] for g in range(CHUNK // 16)]
        for r in range(CHUNK):
            @pl.when(vecs[r // 16][r % 16] == 0)
            def _zero_row():
                for j in range(D // 16):
                    rows_v[r, pl.ds(j * 16, 16)] = zeros

        pltpu.sync_copy(rows_v, out_hbm.at[pl.ds(base + c * CHUNK, CHUNK)])
        return carry

    lax.fori_loop(0, NCH, chunk_body, 0)


@jax.jit
def _sc_embed(x_r, pe):
    mesh = plsc.VectorSubcoreMesh(core_axis_name="c", subcore_axis_name="s")
    return pl.kernel(
        _body,
        out_type=jax.ShapeDtypeStruct((TOTAL, D), jnp.float32),
        mesh=mesh,
        scratch_types=[
            pltpu.VMEM((NCH, CHUNK), jnp.int32),
            pltpu.VMEM((CHUNK, D), jnp.float32),
            pltpu.SemaphoreType.DMA,
        ],
    )(x_r, pe)


def kernel(x, pe):
    out = _sc_embed(x.reshape(NW, NCH, CHUNK), pe)
    return out.reshape(B, S, D)


# speedup vs baseline: 1.7730x; 1.2442x over previous
import jax
import jax.numpy as jnp
from jax import lax
from jax.experimental import pallas as pl
from jax.experimental.pallas import tpu as pltpu
from jax.experimental.pallas import tpu_sc as plsc

B, S = 4, 8192
D = 1024
NC, NS = 2, 16
NW = NC * NS
TOTAL = B * S
PER_W = TOTAL // NW
CHUNK = 32
NCH = PER_W // CHUNK
NBUF = 2
NG = NCH // NBUF


def _body(x_hbm, pe_hbm, out_hbm, idx_v, rows0, rows1, g0, g1, o0, o1):
    rows = (rows0, rows1)
    gsem = (g0, g1)
    osem = (o0, o1)
    wid = lax.axis_index("s") * NC + lax.axis_index("c")
    base = wid * PER_W

    pltpu.sync_copy(x_hbm.at[wid], idx_v)

    zeros = jnp.zeros((16,), jnp.float32)

    def gather(c, b):
        pltpu.make_async_copy(pe_hbm.at[idx_v.at[c]], rows[b], gsem[b]).start()

    def fixup(c, b):
        vecs = [idx_v[c, pl.ds(g * 16, 16)] for g in range(CHUNK // 16)]
        for r in range(CHUNK):
            @pl.when(vecs[r // 16][r % 16] == 0)
            def _zero_row():
                for j in range(D // 16):
                    rows[b][r, pl.ds(j * 16, 16)] = zeros

    def out_start(c, b):
        dst = out_hbm.at[pl.ds(base + c * CHUNK, CHUNK)]
        pltpu.make_async_copy(rows[b], dst, osem[b]).start()

    def out_wait(b):
        dst = out_hbm.at[pl.ds(base, CHUNK)]
        pltpu.make_async_copy(rows[b], dst, osem[b]).wait()

    for b in range(NBUF):
        gather(b, b)

    def outer(g, carry):
        for b in range(NBUF):
            c = g * NBUF + b
            pltpu.make_async_copy(pe_hbm.at[idx_v.at[c]], rows[b], gsem[b]).wait()
            fixup(c, b)
            out_start(c, b)

        @pl.when(g < NG - 1)
        def _issue_next():
            for b in range(NBUF):
                c = g * NBUF + b
                out_wait(b)
                gather(c + NBUF, b)

        return carry

    lax.fori_loop(0, NG, outer, 0)

    for b in range(NBUF):
        out_wait(b)


@jax.jit
def _sc_embed(x_r, pe):
    mesh = plsc.VectorSubcoreMesh(core_axis_name="c", subcore_axis_name="s")
    return pl.kernel(
        _body,
        out_type=jax.ShapeDtypeStruct((TOTAL, D), jnp.float32),
        mesh=mesh,
        scratch_types=[
            pltpu.VMEM((NCH, CHUNK), jnp.int32),
            pltpu.VMEM((CHUNK, D), jnp.float32),
            pltpu.VMEM((CHUNK, D), jnp.float32),
            pltpu.SemaphoreType.DMA,
            pltpu.SemaphoreType.DMA,
            pltpu.SemaphoreType.DMA,
            pltpu.SemaphoreType.DMA,
        ],
    )(x_r, pe)


def kernel(x, pe):
    out = _sc_embed(x.reshape(NW, NCH, CHUNK), pe)
    return out.reshape(B, S, D)


# speedup vs baseline: 2.0858x; 1.1764x over previous
import jax
import jax.numpy as jnp
from jax import lax
from jax.experimental import pallas as pl
from jax.experimental.pallas import tpu as pltpu
from jax.experimental.pallas import tpu_sc as plsc

B, S = 4, 8192
D = 1024
NC, NS = 2, 16
NW = NC * NS
TOTAL = B * S
PER_W = TOTAL // NW
CHUNK = 32
NCH = PER_W // CHUNK
NBUF = 2
NG = NCH // NBUF


def _body(x_hbm, pe_hbm, out_hbm, idx_v, rows0, rows1, g0, g1, o0, o1):
    rows = (rows0, rows1)
    gsem = (g0, g1)
    osem = (o0, o1)
    wid = lax.axis_index("s") * NC + lax.axis_index("c")
    base = wid * PER_W

    pltpu.sync_copy(x_hbm.at[wid], idx_v)

    zeros = jnp.zeros((16,), jnp.float32)

    def gather(c, b):
        pltpu.make_async_copy(pe_hbm.at[idx_v.at[c]], rows[b], gsem[b]).start()

    def fixup(c, b):
        vecs = [idx_v[c, pl.ds(g * 16, 16)] for g in range(CHUNK // 16)]
        for r in range(CHUNK):
            @pl.when(vecs[r // 16][r % 16] == 0)
            def _zero_row():
                for j in range(D // 16):
                    rows[b][r, pl.ds(j * 16, 16)] = zeros

    def out_start(c, b):
        dst = out_hbm.at[pl.ds(base + c * CHUNK, CHUNK)]
        pltpu.make_async_copy(rows[b], dst, osem[b]).start()

    def out_wait(b):
        dst = out_hbm.at[pl.ds(base, CHUNK)]
        pltpu.make_async_copy(rows[b], dst, osem[b]).wait()

    def gather_wait(c, b):
        pltpu.make_async_copy(pe_hbm.at[idx_v.at[c]], rows[b], gsem[b]).wait()

    gather(0, 0)

    def outer(g, carry):
        c = g * 2
        gather_wait(c, 0)

        @pl.when(g > 0)
        def _w():
            out_wait(1)

        gather(c + 1, 1)
        fixup(c, 0)
        out_start(c, 0)

        gather_wait(c + 1, 1)
        out_wait(0)

        @pl.when(g < NG - 1)
        def _n():
            gather(c + 2, 0)

        fixup(c + 1, 1)
        out_start(c + 1, 1)
        return carry

    lax.fori_loop(0, NG, outer, 0)
    out_wait(1)


@jax.jit
def _sc_embed(x_r, pe):
    mesh = plsc.VectorSubcoreMesh(core_axis_name="c", subcore_axis_name="s")
    return pl.kernel(
        _body,
        out_type=jax.ShapeDtypeStruct((TOTAL, D), jnp.float32),
        mesh=mesh,
        scratch_types=[
            pltpu.VMEM((NCH, CHUNK), jnp.int32),
            pltpu.VMEM((CHUNK, D), jnp.float32),
            pltpu.VMEM((CHUNK, D), jnp.float32),
            pltpu.SemaphoreType.DMA,
            pltpu.SemaphoreType.DMA,
            pltpu.SemaphoreType.DMA,
            pltpu.SemaphoreType.DMA,
        ],
    )(x_r, pe)


def kernel(x, pe):
    out = _sc_embed(x.reshape(NW, NCH, CHUNK), pe)
    return out.reshape(B, S, D)


# speedup vs baseline: 2.2571x; 1.0821x over previous
import jax
import jax.numpy as jnp
from jax import lax
from jax.experimental import pallas as pl
from jax.experimental.pallas import tpu as pltpu
from jax.experimental.pallas import tpu_sc as plsc

B, S = 4, 8192
D = 1024
NC, NS = 2, 16
NW = NC * NS
TOTAL = B * S
PER_W = TOTAL // NW
CHUNK = 16
NCH = PER_W // CHUNK
NBUF = 4
NG = NCH // NBUF


def _body(x_hbm, pe_hbm, out_hbm, idx_v,
          rows0, rows1, rows2, rows3,
          g0, g1, g2, g3, o0, o1, o2, o3):
    rows = (rows0, rows1, rows2, rows3)
    gsem = (g0, g1, g2, g3)
    osem = (o0, o1, o2, o3)
    wid = lax.axis_index("s") * NC + lax.axis_index("c")
    base = wid * PER_W

    pltpu.sync_copy(x_hbm.at[wid], idx_v)

    zeros = jnp.zeros((16,), jnp.float32)

    def gather(c, b):
        pltpu.make_async_copy(pe_hbm.at[idx_v.at[c]], rows[b], gsem[b]).start()

    def gather_wait(c, b):
        pltpu.make_async_copy(pe_hbm.at[idx_v.at[c]], rows[b], gsem[b]).wait()

    def fixup(c, b):
        vec = idx_v[c, pl.ds(0, 16)]
        for r in range(CHUNK):
            @pl.when(vec[r] == 0)
            def _zero_row():
                for j in range(D // 16):
                    rows[b][r, pl.ds(j * 16, 16)] = zeros

    def out_start(c, b):
        dst = out_hbm.at[pl.ds(base + c * CHUNK, CHUNK)]
        pltpu.make_async_copy(rows[b], dst, osem[b]).start()

    def out_wait(b):
        dst = out_hbm.at[pl.ds(base, CHUNK)]
        pltpu.make_async_copy(rows[b], dst, osem[b]).wait()

    for b in range(NBUF - 1):
        gather(b, b)

    def outer(g, carry):
        for b in range(NBUF):
            c = g * NBUF + b
            gather_wait(c, b)
            fixup(c, b)
            out_start(c, b)
            bt = (b + NBUF - 1) % NBUF
            if b == 0:
                @pl.when(g > 0)
                def _w0():
                    out_wait(bt)

                gather(c + NBUF - 1, bt)
            else:
                @pl.when(g < NG - 1)
                def _wn():
                    out_wait(bt)
                    gather(c + NBUF - 1, bt)

        return carry

    lax.fori_loop(0, NG, outer, 0)

    for b in range(NBUF):
        out_wait(b)


@jax.jit
def _sc_embed(x_r, pe):
    mesh = plsc.VectorSubcoreMesh(core_axis_name="c", subcore_axis_name="s")
    return pl.kernel(
        _body,
        out_type=jax.ShapeDtypeStruct((TOTAL, D), jnp.float32),
        mesh=mesh,
        scratch_types=[
            pltpu.VMEM((NCH, CHUNK), jnp.int32),
            pltpu.VMEM((CHUNK, D), jnp.float32),
            pltpu.VMEM((CHUNK, D), jnp.float32),
            pltpu.VMEM((CHUNK, D), jnp.float32),
            pltpu.VMEM((CHUNK, D), jnp.float32),
            pltpu.SemaphoreType.DMA,
            pltpu.SemaphoreType.DMA,
            pltpu.SemaphoreType.DMA,
            pltpu.SemaphoreType.DMA,
            pltpu.SemaphoreType.DMA,
            pltpu.SemaphoreType.DMA,
            pltpu.SemaphoreType.DMA,
            pltpu.SemaphoreType.DMA,
        ],
    )(x_r, pe)


def kernel(x, pe):
    out = _sc_embed(x.reshape(NW, NCH, CHUNK), pe)
    return out.reshape(B, S, D)


# speedup vs baseline: 2.3952x; 1.0612x over previous
import jax
import jax.numpy as jnp
from jax import lax
from jax.experimental import pallas as pl
from jax.experimental.pallas import tpu as pltpu
from jax.experimental.pallas import tpu_sc as plsc

B, S = 4, 8192
D = 1024
NC, NS = 2, 16
NW = NC * NS
TOTAL = B * S
PER_W = TOTAL // NW
CHUNK = 16
NCH = PER_W // CHUNK
NBUF = 4
NG = NCH // NBUF


def _body(x_hbm, pe_hbm, out_hbm, idx_v,
          rows0, rows1, rows2, rows3,
          g0, g1, g2, g3, o0, o1, o2, o3):
    rows = (rows0, rows1, rows2, rows3)
    gsem = (g0, g1, g2, g3)
    osem = (o0, o1, o2, o3)
    wid = lax.axis_index("s") * NC + lax.axis_index("c")
    base = wid * PER_W

    pltpu.sync_copy(x_hbm.at[wid], idx_v)

    zeros = jnp.zeros((16,), jnp.float32)

    def gather(c, b):
        pltpu.make_async_copy(pe_hbm.at[idx_v.at[c]], rows[b], gsem[b]).start()

    def gather_wait(c, b):
        pltpu.make_async_copy(pe_hbm.at[idx_v.at[c]], rows[b], gsem[b]).wait()

    def fixup(c, b):
        vec = idx_v[c, pl.ds(0, 16)]
        flag = vec[0] == 0
        for r in range(1, CHUNK):
            flag = jnp.logical_or(flag, vec[r] == 0)

        @pl.when(flag)
        def _slow():
            for r in range(CHUNK):
                @pl.when(vec[r] == 0)
                def _zero_row():
                    for j in range(D // 16):
                        rows[b][r, pl.ds(j * 16, 16)] = zeros

    def out_start(c, b):
        dst = out_hbm.at[pl.ds(base + c * CHUNK, CHUNK)]
        pltpu.make_async_copy(rows[b], dst, osem[b]).start()

    def out_wait(b):
        dst = out_hbm.at[pl.ds(base, CHUNK)]
        pltpu.make_async_copy(rows[b], dst, osem[b]).wait()

    for b in range(NBUF - 1):
        gather(b, b)

    def outer(g, carry):
        for b in range(NBUF):
            c = g * NBUF + b
            gather_wait(c, b)
            fixup(c, b)
            out_start(c, b)
            bt = (b + NBUF - 1) % NBUF
            if b == 0:
                @pl.when(g > 0)
                def _w0():
                    out_wait(bt)

                gather(c + NBUF - 1, bt)
            else:
                @pl.when(g < NG - 1)
                def _wn():
                    out_wait(bt)
                    gather(c + NBUF - 1, bt)

        return carry

    lax.fori_loop(0, NG, outer, 0)

    for b in range(NBUF):
        out_wait(b)


@jax.jit
def _sc_embed(x_r, pe):
    mesh = plsc.VectorSubcoreMesh(core_axis_name="c", subcore_axis_name="s")
    return pl.kernel(
        _body,
        out_type=jax.ShapeDtypeStruct((TOTAL, D), jnp.float32),
        mesh=mesh,
        scratch_types=[
            pltpu.VMEM((NCH, CHUNK), jnp.int32),
            pltpu.VMEM((CHUNK, D), jnp.float32),
            pltpu.VMEM((CHUNK, D), jnp.float32),
            pltpu.VMEM((CHUNK, D), jnp.float32),
            pltpu.VMEM((CHUNK, D), jnp.float32),
            pltpu.SemaphoreType.DMA,
            pltpu.SemaphoreType.DMA,
            pltpu.SemaphoreType.DMA,
            pltpu.SemaphoreType.DMA,
            pltpu.SemaphoreType.DMA,
            pltpu.SemaphoreType.DMA,
            pltpu.SemaphoreType.DMA,
            pltpu.SemaphoreType.DMA,
        ],
    )(x_r, pe)


def kernel(x, pe):
    out = _sc_embed(x.reshape(NW, NCH, CHUNK), pe)
    return out.reshape(B, S, D)


# speedup vs baseline: 2.3953x; 1.0001x over previous
import jax
import jax.numpy as jnp
from jax import lax
from jax.experimental import pallas as pl
from jax.experimental.pallas import tpu as pltpu
from jax.experimental.pallas import tpu_sc as plsc

B, S = 4, 8192
D = 1024
NC, NS = 2, 16
NW = NC * NS
TOTAL = B * S
PER_W = TOTAL // NW
CHUNK = 16
NCH = PER_W // CHUNK
NBUF = 4
NG = NCH // NBUF


def _body(x_hbm, pe_hbm, out_hbm, idx_v,
          rows0, rows1, rows2, rows3,
          g0, g1, g2, g3, o0, o1, o2, o3):
    rows = (rows0, rows1, rows2, rows3)
    gsem = (g0, g1, g2, g3)
    osem = (o0, o1, o2, o3)
    wid = lax.axis_index("s") * NC + lax.axis_index("c")
    base = wid * PER_W

    pltpu.sync_copy(x_hbm.at[wid], idx_v)

    zeros = jnp.zeros((16,), jnp.float32)

    def gather(c, b):
        pltpu.make_async_copy(pe_hbm.at[idx_v.at[c]], rows[b], gsem[b]).start()

    def gather_wait(c, b):
        pltpu.make_async_copy(pe_hbm.at[idx_v.at[c]], rows[b], gsem[b]).wait()

    def detect(c):
        vec = idx_v[c, pl.ds(0, 16)]
        flag = vec[0] == 0
        for r in range(1, CHUNK):
            flag = jnp.logical_or(flag, vec[r] == 0)
        return vec, flag

    def fixup(vec, flag, b):
        @pl.when(flag)
        def _slow():
            for r in range(CHUNK):
                @pl.when(vec[r] == 0)
                def _zero_row():
                    for j in range(D // 16):
                        rows[b][r, pl.ds(j * 16, 16)] = zeros

    def out_start(c, b):
        dst = out_hbm.at[pl.ds(base + c * CHUNK, CHUNK)]
        pltpu.make_async_copy(rows[b], dst, osem[b]).start()

    def out_wait(b):
        dst = out_hbm.at[pl.ds(base, CHUNK)]
        pltpu.make_async_copy(rows[b], dst, osem[b]).wait()

    for b in range(NBUF - 1):
        gather(b, b)

    def outer(g, carry):
        for b in range(NBUF):
            c = g * NBUF + b
            vec, flag = detect(c)
            gather_wait(c, b)
            fixup(vec, flag, b)
            out_start(c, b)
            bt = (b + NBUF - 1) % NBUF
            if b == 0:
                @pl.when(g > 0)
                def _w0():
                    out_wait(bt)

                gather(c + NBUF - 1, bt)
            else:
                @pl.when(g < NG - 1)
                def _wn():
                    out_wait(bt)
                    gather(c + NBUF - 1, bt)

        return carry

    lax.fori_loop(0, NG, outer, 0)

    for b in range(NBUF):
        out_wait(b)


@jax.jit
def _sc_embed(x_r, pe):
    mesh = plsc.VectorSubcoreMesh(core_axis_name="c", subcore_axis_name="s")
    return pl.kernel(
        _body,
        out_type=jax.ShapeDtypeStruct((TOTAL, D), jnp.float32),
        mesh=mesh,
        scratch_types=[
            pltpu.VMEM((NCH, CHUNK), jnp.int32),
            pltpu.VMEM((CHUNK, D), jnp.float32),
            pltpu.VMEM((CHUNK, D), jnp.float32),
            pltpu.VMEM((CHUNK, D), jnp.float32),
            pltpu.VMEM((CHUNK, D), jnp.float32),
            pltpu.SemaphoreType.DMA,
            pltpu.SemaphoreType.DMA,
            pltpu.SemaphoreType.DMA,
            pltpu.SemaphoreType.DMA,
            pltpu.SemaphoreType.DMA,
            pltpu.SemaphoreType.DMA,
            pltpu.SemaphoreType.DMA,
            pltpu.SemaphoreType.DMA,
        ],
    )(x_r, pe)


def kernel(x, pe):
    out = _sc_embed(x.reshape(NW, NCH, CHUNK), pe)
    return out.reshape(B, S, D)


# speedup vs baseline: 2.3960x; 1.0003x over previous
import jax
import jax.numpy as jnp
from jax import lax
from jax.experimental import pallas as pl
from jax.experimental.pallas import tpu as pltpu
from jax.experimental.pallas import tpu_sc as plsc

B, S = 4, 8192
D = 1024
NC, NS = 2, 16
NW = NC * NS
TOTAL = B * S
PER_W = TOTAL // NW
CHUNK = 16
NCH = PER_W // CHUNK
NBUF = 4
NG = NCH // NBUF


def _body(x_hbm, pe_hbm, out_hbm, idx_v,
          rows0, rows1, rows2, rows3,
          g0, g1, g2, g3, o0, o1, o2, o3):
    rows = (rows0, rows1, rows2, rows3)
    gsem = (g0, g1, g2, g3)
    osem = (o0, o1, o2, o3)
    wid = lax.axis_index("s") * NC + lax.axis_index("c")
    base = wid * PER_W

    pltpu.sync_copy(x_hbm.at[wid], idx_v)

    zeros = jnp.zeros((16,), jnp.float32)

    def gather(c, b):
        pltpu.make_async_copy(pe_hbm.at[idx_v.at[c]], rows[b], gsem[b]).start()

    def gather_wait(c, b):
        pltpu.make_async_copy(pe_hbm.at[idx_v.at[c]], rows[b], gsem[b]).wait()

    def detect(c):
        vec = idx_v[c, pl.ds(0, 16)]
        vals = [vec[r] for r in range(CHUNK)]
        flag = vals[0] == 0
        for r in range(1, CHUNK):
            flag = jnp.logical_or(flag, vals[r] == 0)
        return vals, flag

    def fixup(vals, flag, b):
        @pl.when(flag)
        def _slow():
            for r in range(CHUNK):
                @pl.when(vals[r] == 0)
                def _zero_row():
                    for j in range(D // 16):
                        rows[b][r, pl.ds(j * 16, 16)] = zeros

    def out_start(c, b):
        dst = out_hbm.at[pl.ds(base + c * CHUNK, CHUNK)]
        pltpu.make_async_copy(rows[b], dst, osem[b]).start()

    def out_wait(b):
        dst = out_hbm.at[pl.ds(base, CHUNK)]
        pltpu.make_async_copy(rows[b], dst, osem[b]).wait()

    for b in range(NBUF - 1):
        gather(b, b)

    def outer(g, carry):
        for b in range(NBUF):
            c = g * NBUF + b
            vals, flag = detect(c)
            gather_wait(c, b)
            fixup(vals, flag, b)
            out_start(c, b)
            bt = (b + NBUF - 1) % NBUF
            if b == 0:
                @pl.when(g > 0)
                def _w0():
                    out_wait(bt)

                gather(c + NBUF - 1, bt)
            else:
                @pl.when(g < NG - 1)
                def _wn():
                    out_wait(bt)
                    gather(c + NBUF - 1, bt)

        return carry

    lax.fori_loop(0, NG, outer, 0)

    for b in range(NBUF):
        out_wait(b)


@jax.jit
def _sc_embed(x_r, pe):
    mesh = plsc.VectorSubcoreMesh(core_axis_name="c", subcore_axis_name="s")
    return pl.kernel(
        _body,
        out_type=jax.ShapeDtypeStruct((TOTAL, D), jnp.float32),
        mesh=mesh,
        scratch_types=[
            pltpu.VMEM((NCH, CHUNK), jnp.int32),
            pltpu.VMEM((CHUNK, D), jnp.float32),
            pltpu.VMEM((CHUNK, D), jnp.float32),
            pltpu.VMEM((CHUNK, D), jnp.float32),
            pltpu.VMEM((CHUNK, D), jnp.float32),
            pltpu.SemaphoreType.DMA,
            pltpu.SemaphoreType.DMA,
            pltpu.SemaphoreType.DMA,
            pltpu.SemaphoreType.DMA,
            pltpu.SemaphoreType.DMA,
            pltpu.SemaphoreType.DMA,
            pltpu.SemaphoreType.DMA,
            pltpu.SemaphoreType.DMA,
        ],
    )(x_r, pe)


def kernel(x, pe):
    out = _sc_embed(x.reshape(NW, NCH, CHUNK), pe)
    return out.reshape(B, S, D)
